# Initial kernel scaffold; baseline (speedup 1.0000x reference)
import jax
import jax.numpy as jnp
from jax.experimental import pallas as pl


def kernel(inputs, table):
    raise NotImplementedError("write your pallas kernel here")


# speedup vs baseline: 1.4991x; 1.4991x over previous
import functools

import jax
import jax.numpy as jnp
from jax.experimental import pallas as pl


def _add_kernel(in_ref, tab_ref, out_ref):
    out_ref[...] = in_ref[...] + tab_ref[...]


@functools.partial(jax.jit, static_argnames=("block",))
def _posemb_add(inputs, table, block=512):
    batch, seq, dim = inputs.shape
    grid = (seq // block, batch)
    return pl.pallas_call(
        _add_kernel,
        grid=grid,
        in_specs=[
            pl.BlockSpec((1, block, dim), lambda s, b: (b, s, 0)),
            pl.BlockSpec((block, dim), lambda s, b: (s, 0)),
        ],
        out_specs=pl.BlockSpec((1, block, dim), lambda s, b: (b, s, 0)),
        out_shape=jax.ShapeDtypeStruct(inputs.shape, inputs.dtype),
    )(inputs, table)


def kernel(inputs, table):
    return _posemb_add(inputs, table)


# speedup vs baseline: 1.6700x; 1.1140x over previous
import functools

import jax
import jax.numpy as jnp
from jax.experimental import pallas as pl


def _add_kernel(in_ref, tab_ref, out_ref):
    out_ref[...] = in_ref[...] + tab_ref[...]


@functools.partial(jax.jit, static_argnames=("block",))
def _posemb_add(inputs, table, block=1024):
    batch, seq, dim = inputs.shape
    grid = (seq // block, batch)
    return pl.pallas_call(
        _add_kernel,
        grid=grid,
        in_specs=[
            pl.BlockSpec((1, block, dim), lambda s, b: (b, s, 0)),
            pl.BlockSpec((block, dim), lambda s, b: (s, 0)),
        ],
        out_specs=pl.BlockSpec((1, block, dim), lambda s, b: (b, s, 0)),
        out_shape=jax.ShapeDtypeStruct(inputs.shape, inputs.dtype),
    )(inputs, table)


def kernel(inputs, table):
    return _posemb_add(inputs, table)


# speedup vs baseline: 1.7366x; 1.0399x over previous
import functools

import jax
import jax.numpy as jnp
from jax.experimental import pallas as pl


def _add_kernel(in_ref, tab_ref, out_ref):
    out_ref[...] = in_ref[...] + tab_ref[...]


@functools.partial(jax.jit, static_argnames=("block",))
def _posemb_add(inputs, table, block=2048):
    batch, seq, dim = inputs.shape
    grid = (seq // block, batch)
    return pl.pallas_call(
        _add_kernel,
        grid=grid,
        in_specs=[
            pl.BlockSpec((1, block, dim), lambda s, b: (b, s, 0)),
            pl.BlockSpec((block, dim), lambda s, b: (s, 0)),
        ],
        out_specs=pl.BlockSpec((1, block, dim), lambda s, b: (b, s, 0)),
        out_shape=jax.ShapeDtypeStruct(inputs.shape, inputs.dtype),
    )(inputs, table)


def kernel(inputs, table):
    return _posemb_add(inputs, table)
